# Initial kernel scaffold; baseline (speedup 1.0000x reference)
import jax
import jax.numpy as jnp
from jax.experimental import pallas as pl


def kernel(x, c0w, c0b, c1w, c1b, c2w, c2b, c3w, c3b, c4w, c4b, gwi, gwh, gbi, gbh, mw1, mb1, mw2, mb2):
    raise NotImplementedError("write your pallas kernel here")


# speedup vs baseline: 22.6882x; 22.6882x over previous
import functools

import jax
import jax.numpy as jnp
from jax.experimental import pallas as pl
from jax.experimental.pallas import tpu as pltpu


_CHANNELS = ((1, 8), (8, 16), (16, 32), (32, 64), (64, 128))


def _dims(length):
    dims = []
    t = length
    for _ in _CHANNELS:
        t_conv = t - 2
        t_pool = t_conv // 3
        dims.append((t_conv, t_pool))
        t = t_pool
    return tuple(dims), t


def _body(stage_dims, t_gru, bb,
          x_ref,
          w1, b1, w2, b2, w3, b3, w4, b4, w5, b5,
          wiT, whT, biT, bhT,
          wl1T, bl1T, wl2T, bl2T,
          o_ref,
          h1_ref, h2_ref, h3_ref, h4_ref):
    f32 = jnp.float32
    conv_w = (w2, w3, w4, w5)
    conv_b = (b2, b3, b4, b5)
    h_refs = (h1_ref, h2_ref, h3_ref, h4_ref)

    tp1 = stage_dims[0][1]
    g0 = x_ref[pl.ds(0, tp1 + 1, 3), :]
    g1 = x_ref[pl.ds(1, tp1 + 1, 3), :]
    g2 = x_ref[pl.ds(2, tp1, 3), :]
    xs = jnp.stack([g0[:tp1], g1[:tp1], g2, g0[1:], g1[1:]], axis=0)
    p = None
    for r in range(3):
        c = jnp.einsum("oj,jtb->otb", w1[...], xs[r:r + 3],
                       preferred_element_type=f32)
        p = c if p is None else jnp.maximum(p, c)
    h1_ref[...] = jnp.maximum(p + b1[...], 0.0)

    for s in range(1, 5):
        r_prev = h_refs[s - 1]
        cin = _CHANNELS[s][0]
        tp = stage_dims[s][1]
        g0 = r_prev[:, pl.ds(0, tp + 1, 3), :]
        g1 = r_prev[:, pl.ds(1, tp + 1, 3), :]
        g2 = r_prev[:, pl.ds(2, tp, 3), :]
        hs = jnp.concatenate(
            [g0[:, :tp], g1[:, :tp], g2, g0[:, 1:], g1[:, 1:]], axis=0)
        p = None
        for r in range(3):
            c = jnp.einsum("oj,jtb->otb", conv_w[s - 1][...],
                           hs[r * cin:(r + 3) * cin],
                           preferred_element_type=f32)
            p = c if p is None else jnp.maximum(p, c)
        h = jnp.maximum(p + conv_b[s - 1][...], 0.0)
        if s < 4:
            h_refs[s][...] = h
        else:
            h5 = h

    hdim = 32
    xts = [h5[:, t, :] for t in range(t_gru)]

    def run_direction(d, inputs):
        h = jnp.zeros((hdim, bb), f32)
        h_max = None
        h_sum = None
        for x_t in inputs:
            gi = jnp.dot(wiT[d], x_t, preferred_element_type=f32) + biT[d]
            gh = jnp.dot(whT[d], h, preferred_element_type=f32) + bhT[d]
            r = jax.nn.sigmoid(gi[0:hdim] + gh[0:hdim])
            z = jax.nn.sigmoid(gi[hdim:2 * hdim] + gh[hdim:2 * hdim])
            n = jnp.tanh(gi[2 * hdim:] + r * gh[2 * hdim:])
            h = (1.0 - z) * n + z * h
            h_max = h if h_max is None else jnp.maximum(h_max, h)
            h_sum = h if h_sum is None else h_sum + h
        return h, h_max, h_sum * (1.0 / len(inputs))

    h_f, max_f, mean_f = run_direction(0, xts)
    h_b, max_b, mean_b = run_direction(1, xts[::-1])

    feat = jnp.concatenate([max_f, max_b, mean_f, mean_b, h_f, h_b], axis=0)
    hid = jnp.dot(wl1T[...], feat, preferred_element_type=f32) + bl1T[...]
    hid = jnp.maximum(hid, 0.0)
    logit = jnp.dot(wl2T[...], hid, preferred_element_type=f32) + bl2T[...]
    o_ref[...] = jax.nn.sigmoid(logit)


def kernel(x, c0w, c0b, c1w, c1b, c2w, c2b, c3w, c3b, c4w, c4b,
           gwi, gwh, gbi, gbh, mw1, mb1, mw2, mb2):
    batch, _, length = x.shape
    stage_dims, t_gru = _dims(length)

    bb = 128
    g = pl.cdiv(batch, bb)
    xT = jnp.transpose(x.reshape(batch, length))
    if g * bb != batch:
        xT = jnp.pad(xT, ((0, 0), (0, g * bb - batch)))

    conv_ws = []
    for w, b in ((c0w, c0b), (c1w, c1b), (c2w, c2b), (c3w, c3b), (c4w, c4b)):
        conv_ws += [jnp.transpose(w), jnp.transpose(b)[:, :, None]]
    wiT = jnp.swapaxes(gwi, 1, 2)
    whT = jnp.swapaxes(gwh, 1, 2)
    biT = jnp.swapaxes(gbi, 1, 2)
    bhT = jnp.swapaxes(gbh, 1, 2)
    wl1T = jnp.transpose(mw1)
    bl1T = jnp.transpose(mb1)
    wl2T = jnp.transpose(mw2)
    bl2T = mb2
    weights = conv_ws + [wiT, whT, biT, bhT, wl1T, bl1T, wl2T, bl2T]

    def _const_spec(a):
        return pl.BlockSpec(a.shape, lambda i, _n=a.ndim: (0,) * _n)

    in_specs = [pl.BlockSpec((length, bb), lambda i: (0, i))]
    in_specs += [_const_spec(a) for a in weights]

    scratch_shapes = [
        pltpu.VMEM((c_out, t_pool, bb), jnp.float32)
        for (_, t_pool), (_, c_out) in zip(stage_dims[:4], _CHANNELS[:4])
    ]

    body = functools.partial(_body, stage_dims, t_gru, bb)
    out = pl.pallas_call(
        body,
        out_shape=jax.ShapeDtypeStruct((1, g * bb), jnp.float32),
        grid=(g,),
        in_specs=in_specs,
        out_specs=pl.BlockSpec((1, bb), lambda i: (0, i)),
        scratch_shapes=scratch_shapes,
        compiler_params=pltpu.CompilerParams(
            dimension_semantics=("parallel",)),
    )(xT, *weights)
    return out.reshape(g * bb, 1)[:batch]


# speedup vs baseline: 43.8784x; 1.9340x over previous
import functools

import jax
import jax.numpy as jnp
from jax.experimental import pallas as pl
from jax.experimental.pallas import tpu as pltpu


_CHANNELS = ((1, 8), (8, 16), (16, 32), (32, 64), (64, 128))


def _dims(length):
    dims = []
    t = length
    for _ in _CHANNELS:
        t_conv = t - 2
        t_pool = t_conv // 3
        dims.append((t_conv, t_pool))
        t = t_pool
    return tuple(dims), t


def _body(stage_dims, t_gru, bb, length,
          x_ref,
          w1big, b1big, w2, b2, w3, b3, w4, b4, w5, b5,
          wiT, whT, biT, bhT,
          wl1T, bl1T, wl2T, bl2T,
          o_ref):
    f32 = jnp.float32

    tp1 = stage_dims[0][1]
    xall = x_ref[...]
    nt = length // 8
    tiles = [xall[8 * t:8 * t + 8, :] for t in range(nt)]
    nj = (tp1 + 7) // 8
    wins = []
    for J in range(nj):
        idx = [min(3 * J + d, nt - 1) for d in range(4)]
        wins.append(jnp.concatenate([tiles[i] for i in idx], axis=0))
    x1 = jnp.concatenate(wins, axis=1)
    w1 = w1big[...]
    p = None
    for r in range(3):
        c = jnp.dot(w1[r], x1, preferred_element_type=f32)
        p = c if p is None else jnp.maximum(p, c)
    h = jnp.maximum(p + b1big[...], 0.0)
    planes = [h[8 * (j % 8):8 * (j % 8) + 8,
               (j // 8) * bb:(j // 8 + 1) * bb] for j in range(tp1)]

    for s, (w_s, b_s) in enumerate(((w2, b2), (w3, b3), (w4, b4), (w5, b5)),
                                   start=1):
        tp = stage_dims[s][1]
        gm = [jnp.concatenate([planes[3 * j + m] for j in range(tp)], axis=1)
              for m in range(5)]
        w = w_s[...]
        p = None
        for r in range(3):
            xr = jnp.concatenate([gm[r], gm[r + 1], gm[r + 2]], axis=0)
            c = jnp.dot(w, xr, preferred_element_type=f32)
            p = c if p is None else jnp.maximum(p, c)
        h = jnp.maximum(p + b_s[...], 0.0)
        planes = [h[:, j * bb:(j + 1) * bb] for j in range(tp)]

    hdim = 32
    xts = planes

    def run_direction(d, inputs):
        hh = jnp.zeros((hdim, bb), f32)
        h_max = None
        h_sum = None
        for x_t in inputs:
            gi = jnp.dot(wiT[d], x_t, preferred_element_type=f32) + biT[d]
            gh = jnp.dot(whT[d], hh, preferred_element_type=f32) + bhT[d]
            r = jax.nn.sigmoid(gi[0:hdim] + gh[0:hdim])
            z = jax.nn.sigmoid(gi[hdim:2 * hdim] + gh[hdim:2 * hdim])
            n = jnp.tanh(gi[2 * hdim:] + r * gh[2 * hdim:])
            hh = (1.0 - z) * n + z * hh
            h_max = hh if h_max is None else jnp.maximum(h_max, hh)
            h_sum = hh if h_sum is None else h_sum + hh
        return hh, h_max, h_sum * (1.0 / len(inputs))

    h_f, max_f, mean_f = run_direction(0, xts)
    h_b, max_b, mean_b = run_direction(1, xts[::-1])

    feat = jnp.concatenate([max_f, max_b, mean_f, mean_b, h_f, h_b], axis=0)
    hid = jnp.dot(wl1T[...], feat, preferred_element_type=f32) + bl1T[...]
    hid = jnp.maximum(hid, 0.0)
    logit = jnp.dot(wl2T[...], hid, preferred_element_type=f32) + bl2T[...]
    o_ref[...] = jax.nn.sigmoid(logit)


def kernel(x, c0w, c0b, c1w, c1b, c2w, c2b, c3w, c3b, c4w, c4b,
           gwi, gwh, gbi, gbh, mw1, mb1, mw2, mb2):
    batch, _, length = x.shape
    stage_dims, t_gru = _dims(length)

    bb = 128
    g = pl.cdiv(batch, bb)
    xT = jnp.transpose(x.reshape(batch, length))
    if g * bb != batch:
        xT = jnp.pad(xT, ((0, 0), (0, g * bb - batch)))

    c0wT = jnp.transpose(c0w)
    w1big = jnp.stack([
        jnp.concatenate(
            [jnp.pad(c0wT, ((0, 0), (3 * u + r, 32 - 3 * u - r - 3)))
             for u in range(8)], axis=0)
        for r in range(3)])
    b1big = jnp.tile(jnp.transpose(c0b), (8, 1))
    conv_ws = [w1big, b1big]
    for w, b in ((c1w, c1b), (c2w, c2b), (c3w, c3b), (c4w, c4b)):
        conv_ws += [jnp.transpose(w), jnp.transpose(b)]
    wiT = jnp.swapaxes(gwi, 1, 2)
    whT = jnp.swapaxes(gwh, 1, 2)
    biT = jnp.swapaxes(gbi, 1, 2)
    bhT = jnp.swapaxes(gbh, 1, 2)
    wl1T = jnp.transpose(mw1)
    bl1T = jnp.transpose(mb1)
    wl2T = jnp.transpose(mw2)
    bl2T = mb2
    weights = conv_ws + [wiT, whT, biT, bhT, wl1T, bl1T, wl2T, bl2T]

    def _const_spec(a):
        return pl.BlockSpec(a.shape, lambda i, _n=a.ndim: (0,) * _n)

    in_specs = [pl.BlockSpec((length, bb), lambda i: (0, i))]
    in_specs += [_const_spec(a) for a in weights]

    body = functools.partial(_body, stage_dims, t_gru, bb, length)
    out = pl.pallas_call(
        body,
        out_shape=jax.ShapeDtypeStruct((1, g * bb), jnp.float32),
        grid=(g,),
        in_specs=in_specs,
        out_specs=pl.BlockSpec((1, bb), lambda i: (0, i)),
        compiler_params=pltpu.CompilerParams(
            dimension_semantics=("parallel",)),
    )(xT, *weights)
    return out.reshape(g * bb, 1)[:batch]
